# baseline (speedup 1.0000x reference)
import functools

import jax
import jax.numpy as jnp
from jax import lax
from jax.experimental import pallas as pl
from jax.experimental.pallas import tpu as pltpu
from jax.experimental.pallas import tpu_sc as plsc

_L = 200
_B = 4096
_V = 1_000_000
_D = 64

_PACK = 8
_KDIM = _PACK * _D
_ROWS = _V // _PACK
_BLK = 5000

_NC, _NS = 2, 16
_NW = _NC * _NS
_BPW = _B // _NW
_LANES = 16


def _matvec_body(x_ref, w_ref, o_ref):
    y = jnp.dot(x_ref[...].astype(jnp.bfloat16), w_ref[...],
                preferred_element_type=jnp.float32)
    o_ref[...] = y.reshape(_BLK * 16 * _PACK)


def _scalarize(embr, wbd):
    grid = _ROWS // _BLK
    return pl.pallas_call(
        _matvec_body,
        grid=(grid,),
        in_specs=[
            pl.BlockSpec((_BLK, _KDIM), lambda i: (i, 0)),
            pl.BlockSpec((_KDIM, 16 * _PACK), lambda i: (0, 0)),
        ],
        out_specs=pl.BlockSpec((_BLK * 16 * _PACK,), lambda i: (i,)),
        out_shape=jax.ShapeDtypeStruct((16 * _V,), jnp.float32),
    )(embr, wbd)


def _sc_pool_body(text_hbm, s_hbm, out_hbm, idx2_v, idx_v, val_v, res_v, sem):
    wid = lax.axis_index("s") * _NC + lax.axis_index("c")
    pltpu.sync_copy(text_hbm.at[:, pl.ds(wid * _BPW, _BPW)], idx2_v)

    def fbody(l, carry):
        for g in range(_BPW // _LANES):
            v = idx2_v[l, pl.ds(g * _LANES, _LANES)]
            idx_v[pl.ds(l * _BPW + g * _LANES, _LANES)] = (
                ((v >> 3) << 7) + (v & 7))
        return carry

    lax.fori_loop(0, _L, fbody, 0)
    pltpu.async_copy(s_hbm.at[idx_v], val_v, sem).wait()
    ngroups = _BPW // _LANES
    zero = jnp.zeros((_LANES,), jnp.float32)

    def body(l, accs):
        row = l * _BPW
        return tuple(accs[g] + val_v[pl.ds(row + g * _LANES, _LANES)]
                     for g in range(ngroups))

    accs = lax.fori_loop(0, _L, body, tuple(zero for _ in range(ngroups)))
    for g in range(ngroups):
        res_v[pl.ds(g * _LANES, _LANES)] = accs[g]
    pltpu.sync_copy(res_v, out_hbm.at[pl.ds(wid * _BPW, _BPW)])


def _sc_pool(text, s):
    mesh = plsc.VectorSubcoreMesh(core_axis_name="c", subcore_axis_name="s")
    k = functools.partial(
        pl.kernel,
        mesh=mesh,
        out_type=jax.ShapeDtypeStruct((_B,), jnp.float32),
        scratch_types=[
            pltpu.VMEM((_L, _BPW), jnp.int32),
            pltpu.VMEM((_L * _BPW,), jnp.int32),
            pltpu.VMEM((_L * _BPW,), jnp.float32),
            pltpu.VMEM((_BPW,), jnp.float32),
            pltpu.SemaphoreType.DMA,
        ],
    )(_sc_pool_body)
    return k(text, s)


def kernel(text, text_len, emb, W, b):
    del text_len
    embr = emb.reshape(_ROWS, _KDIM)
    wbd = jnp.tile(jnp.kron(jnp.eye(_PACK, dtype=jnp.float32),
                            W * (1.0 / _L)), (1, 16)).astype(jnp.bfloat16)
    s = _scalarize(embr, wbd)
    pooled = _sc_pool(text, s)
    return pooled.reshape(_B, 1) + b
